# baseline (speedup 1.0000x reference)
import jax
import jax.numpy as jnp
from jax.experimental import pallas as pl

NUM_EXPERTS = 64
LANES = 128
GRID = 16


def _body(mask_ref, routes_ref, imp_ref, load_ref):
    i = pl.program_id(0)
    rows = mask_ref.shape[0]
    row = jax.lax.broadcasted_iota(jnp.int32, (rows, LANES), 0) + i * rows
    col = jax.lax.broadcasted_iota(jnp.int32, (rows, LANES), 1)
    per_expert_view_rows = (rows * GRID) // NUM_EXPERTS
    blk = ((col & (NUM_EXPERTS - 1)) == (row // per_expert_view_rows)).astype(
        mask_ref.dtype
    )
    mask_ref[...] = blk
    routes_ref[...] = blk
    s = jnp.sum(blk, axis=0)
    s64 = jax.lax.slice(s, (0,), (NUM_EXPERTS,)) + jax.lax.slice(
        s, (NUM_EXPERTS,), (LANES,)
    )

    @pl.when(i == 0)
    def _():
        imp_ref[...] = jnp.zeros_like(imp_ref)
        load_ref[...] = jnp.zeros_like(load_ref)

    imp_ref[...] += s64
    load_ref[...] += s64


def kernel(x):
    n = x.shape[0]
    assert n % (2 * NUM_EXPERTS) == 0 and (n // NUM_EXPERTS) % 2 == 0
    dt = x.dtype
    vrows = n // 2
    brows = vrows // GRID
    out_shape = (
        jax.ShapeDtypeStruct((vrows, LANES), dt),
        jax.ShapeDtypeStruct((vrows, LANES), dt),
        jax.ShapeDtypeStruct((NUM_EXPERTS,), dt),
        jax.ShapeDtypeStruct((NUM_EXPERTS,), dt),
    )
    mask_v, routes_v, imp, load = pl.pallas_call(
        _body,
        grid=(GRID,),
        out_specs=(
            pl.BlockSpec((brows, LANES), lambda i: (i, 0)),
            pl.BlockSpec((brows, LANES), lambda i: (i, 0)),
            pl.BlockSpec((NUM_EXPERTS,), lambda i: (0,)),
            pl.BlockSpec((NUM_EXPERTS,), lambda i: (0,)),
        ),
        out_shape=out_shape,
    )()
    mask = mask_v.reshape(n, NUM_EXPERTS)
    routes = routes_v.reshape(n, NUM_EXPERTS)
    return (mask, routes, imp, load)


# speedup vs baseline: 1.9055x; 1.9055x over previous
import jax
import jax.numpy as jnp
from jax.experimental import pallas as pl
from jax.experimental.pallas import tpu as pltpu

NUM_EXPERTS = 64
GRID = 16


def _body(mask_ref, imp_ref, acc_ref):
    i = pl.program_id(0)
    rows = mask_ref.shape[0]
    row = jax.lax.broadcasted_iota(jnp.int32, (rows, NUM_EXPERTS), 0) + i * rows
    col = jax.lax.broadcasted_iota(jnp.int32, (rows, NUM_EXPERTS), 1)
    per_expert = (rows * GRID) // NUM_EXPERTS
    blk = (col == (row // per_expert)).astype(mask_ref.dtype)
    mask_ref[...] = blk

    @pl.when(i == 0)
    def _():
        acc_ref[...] = jnp.zeros_like(acc_ref)

    acc_ref[...] += jnp.sum(blk, axis=0)

    @pl.when(i == GRID - 1)
    def _():
        imp_ref[...] = acc_ref[...]


def kernel(x):
    n = x.shape[0]
    assert n % NUM_EXPERTS == 0
    dt = x.dtype
    brows = n // GRID
    mask, imp = pl.pallas_call(
        _body,
        grid=(GRID,),
        out_specs=(
            pl.BlockSpec((brows, NUM_EXPERTS), lambda i: (i, 0)),
            pl.BlockSpec((NUM_EXPERTS,), lambda i: (0,)),
        ),
        out_shape=(
            jax.ShapeDtypeStruct((n, NUM_EXPERTS), dt),
            jax.ShapeDtypeStruct((NUM_EXPERTS,), dt),
        ),
        scratch_shapes=[pltpu.VMEM((NUM_EXPERTS,), dt)],
    )()
    return (mask, mask, imp, imp)


# speedup vs baseline: 2.2926x; 1.2031x over previous
import jax
import jax.numpy as jnp
from jax.experimental import pallas as pl

NUM_EXPERTS = 64


def _body(mask_ref, imp_ref):
    rows = mask_ref.shape[0]
    per = rows // NUM_EXPERTS
    col = jax.lax.broadcasted_iota(jnp.int32, (per, NUM_EXPERTS), 1)

    def step(e, acc):
        pat = (col == e).astype(mask_ref.dtype)
        mask_ref[pl.ds(e * per, per), :] = pat
        return acc + jnp.sum(pat, axis=0)

    acc0 = jnp.zeros((NUM_EXPERTS,), mask_ref.dtype)
    imp_ref[...] = jax.lax.fori_loop(0, NUM_EXPERTS, step, acc0)


def kernel(x):
    n = x.shape[0]
    assert n % NUM_EXPERTS == 0
    dt = x.dtype
    mask, imp = pl.pallas_call(
        _body,
        out_shape=(
            jax.ShapeDtypeStruct((n, NUM_EXPERTS), dt),
            jax.ShapeDtypeStruct((NUM_EXPERTS,), dt),
        ),
    )()
    return (mask, mask, imp, imp)


# speedup vs baseline: 2.6189x; 1.1423x over previous
import jax
import jax.numpy as jnp
from jax.experimental import pallas as pl
from jax.experimental.pallas import tpu as pltpu

NUM_EXPERTS = 64
NCHUNKS = 16


def _body(mask_hbm, imp_ref, buf, sems):
    n = buf.shape[0]
    per = n // NUM_EXPERTS
    ch_rows = n // NCHUNKS
    epc = NUM_EXPERTS // NCHUNKS
    col = jax.lax.broadcasted_iota(jnp.int32, (per, NUM_EXPERTS), 1)
    acc = jnp.zeros((NUM_EXPERTS,), imp_ref.dtype)
    for c in range(NCHUNKS):

        def fill(k, a, c=c):
            e = c * epc + k
            pat = (col == e).astype(buf.dtype)
            buf[pl.ds(c * ch_rows + k * per, per), :] = pat
            return a + jnp.sum(pat, axis=0)

        acc = jax.lax.fori_loop(0, epc, fill, acc)
        pltpu.make_async_copy(
            buf.at[pl.ds(c * ch_rows, ch_rows), :],
            mask_hbm.at[pl.ds(c * ch_rows, ch_rows), :],
            sems.at[c],
        ).start()
    for c in range(NCHUNKS):
        pltpu.make_async_copy(
            buf.at[pl.ds(c * ch_rows, ch_rows), :],
            mask_hbm.at[pl.ds(c * ch_rows, ch_rows), :],
            sems.at[c],
        ).wait()
    imp_ref[...] = acc


def kernel(x):
    n = x.shape[0]
    assert n % NUM_EXPERTS == 0 and NUM_EXPERTS % NCHUNKS == 0
    dt = x.dtype
    mask, imp = pl.pallas_call(
        _body,
        out_shape=(
            jax.ShapeDtypeStruct((n, NUM_EXPERTS), dt),
            jax.ShapeDtypeStruct((NUM_EXPERTS,), dt),
        ),
        out_specs=(
            pl.BlockSpec(memory_space=pltpu.MemorySpace.HBM),
            pl.BlockSpec(memory_space=pltpu.MemorySpace.VMEM),
        ),
        scratch_shapes=[
            pltpu.VMEM((n, NUM_EXPERTS), dt),
            pltpu.SemaphoreType.DMA((NCHUNKS,)),
        ],
    )()
    return (mask, mask, imp, imp)
